# Initial kernel scaffold; baseline (speedup 1.0000x reference)
import jax
import jax.numpy as jnp
from jax.experimental import pallas as pl


def kernel(x, table):
    raise NotImplementedError("write your pallas kernel here")


# speedup vs baseline: 4.3366x; 4.3366x over previous
import jax
import jax.numpy as jnp
from jax import lax
from jax.experimental import pallas as pl
from jax.experimental.pallas import tpu as pltpu
from jax.experimental.pallas import tpu_sc as plsc

VOCAB = 1000
EMB = 16
NC = 2
NS = 16
NW = NC * NS
N = 4096 * 200
PER_W = N // NW
CHUNK = 3200
NCHUNK = PER_W // CHUNK


def _emb_kernel(x_hbm, table_hbm, out_hbm, idx_v, rows_v, sem):
  wid = lax.axis_index("s") * NC + lax.axis_index("c")

  def chunk_body(c, _):
    base = wid * PER_W + c * CHUNK
    pltpu.sync_copy(x_hbm.at[pl.ds(base, CHUNK)], idx_v)
    pltpu.async_copy(table_hbm.at[idx_v], rows_v, sem).wait()

    def relu_body(i, _):
      rows_v[i] = jnp.maximum(rows_v[i], 0.0)
      return ()

    lax.fori_loop(0, CHUNK, relu_body, ())
    pltpu.sync_copy(rows_v, out_hbm.at[pl.ds(base, CHUNK)])
    return ()

  lax.fori_loop(0, NCHUNK, chunk_body, ())


@jax.jit
def _run(x_flat, table):
  mesh = plsc.VectorSubcoreMesh(core_axis_name="c", subcore_axis_name="s")
  return pl.kernel(
      _emb_kernel,
      out_type=jax.ShapeDtypeStruct((N, EMB), jnp.float32),
      mesh=mesh,
      scratch_types=[
          pltpu.VMEM((CHUNK,), jnp.int32),
          pltpu.VMEM((CHUNK, EMB), jnp.float32),
          pltpu.SemaphoreType.DMA,
      ],
      compiler_params=pltpu.CompilerParams(use_tc_tiling_on_sc=False),
  )(x_flat, table)


def kernel(x, table):
  b, h = x.shape
  x_flat = x.reshape(-1).astype(jnp.int32)
  out = _run(x_flat, table)
  return out.reshape(b, h, EMB)


# speedup vs baseline: 5.6089x; 1.2934x over previous
import jax
import jax.numpy as jnp
from jax import lax
from jax.experimental import pallas as pl
from jax.experimental.pallas import tpu as pltpu
from jax.experimental.pallas import tpu_sc as plsc

VOCAB = 1000
EMB = 16
NC = 2
NS = 16
NW = NC * NS
N = 4096 * 200
PER_W = N // NW
CHUNK = 3200
NCHUNK = PER_W // CHUNK


def _emb_kernel(x_hbm, table_hbm, out_hbm, rtab_hbm,
                idx_v0, idx_v1, rows_v0, rows_v1,
                gsem0, gsem1, osem0, osem1):
  wid = lax.axis_index("s") * NC + lax.axis_index("c")
  tb = wid * VOCAB

  pltpu.sync_copy(table_hbm, rows_v0.at[pl.ds(0, VOCAB)])

  @plsc.parallel_loop(0, VOCAB, unroll=8)
  def _(i):
    rows_v0[i] = jnp.maximum(rows_v0[i], 0.0)

  pltpu.sync_copy(rows_v0.at[pl.ds(0, VOCAB)], rtab_hbm.at[pl.ds(tb, VOCAB)])

  bufs = [(idx_v0, rows_v0, gsem0, osem0), (idx_v1, rows_v1, gsem1, osem1)]

  def load_and_gather(c):
    idx_v, rows_v, gsem, _ = bufs[c % 2]
    base = wid * PER_W + c * CHUNK
    pltpu.sync_copy(x_hbm.at[pl.ds(base, CHUNK)], idx_v)

    @plsc.parallel_loop(0, CHUNK // 16, unroll=8)
    def _(j):
      s = pl.ds(j * 16, 16)
      idx_v[s] = idx_v[s] + tb

    return pltpu.async_copy(rtab_hbm.at[idx_v], rows_v, gsem)

  gather = [None] * NCHUNK
  store = [None] * NCHUNK
  gather[0] = load_and_gather(0)
  for c in range(NCHUNK):
    idx_v, rows_v, _, osem = bufs[c % 2]
    if c + 1 < NCHUNK:
      if c - 1 >= 0:
        store[c - 1].wait()
      gather[c + 1] = load_and_gather(c + 1)
    gather[c].wait()
    base = wid * PER_W + c * CHUNK
    store[c] = pltpu.async_copy(rows_v, out_hbm.at[pl.ds(base, CHUNK)], osem)
  store[NCHUNK - 2].wait()
  store[NCHUNK - 1].wait()


@jax.jit
def _run(x_flat, table):
  mesh = plsc.VectorSubcoreMesh(core_axis_name="c", subcore_axis_name="s")
  out, _ = pl.kernel(
      _emb_kernel,
      out_type=(
          jax.ShapeDtypeStruct((N, EMB), jnp.float32),
          jax.ShapeDtypeStruct((NW * VOCAB, EMB), jnp.float32),
      ),
      mesh=mesh,
      scratch_types=[
          pltpu.VMEM((CHUNK,), jnp.int32),
          pltpu.VMEM((CHUNK,), jnp.int32),
          pltpu.VMEM((CHUNK, EMB), jnp.float32),
          pltpu.VMEM((CHUNK, EMB), jnp.float32),
          pltpu.SemaphoreType.DMA,
          pltpu.SemaphoreType.DMA,
          pltpu.SemaphoreType.DMA,
          pltpu.SemaphoreType.DMA,
      ],
      compiler_params=pltpu.CompilerParams(use_tc_tiling_on_sc=False),
  )(x_flat, table)
  return out


def kernel(x, table):
  b, h = x.shape
  x_flat = x.reshape(-1).astype(jnp.int32)
  out = _run(x_flat, table)
  return out.reshape(b, h, EMB)
